# baseline (speedup 1.0000x reference)
import functools

import jax
import jax.numpy as jnp
from jax import lax
from jax.experimental import pallas as pl
from jax.experimental.pallas import tpu as pltpu
from jax.experimental.pallas import tpu_sc as plsc

_N = 10000
_E = 320000
_D = 128
_EPS = 1e-5
_L = 16
_NCH = _D // _L
_NH = _N // 2
_NHP = 5120
_EH = _E // 2
_B = 1280
_NBLK = _EH // _B
_GCH = 128

_NP = 10016
_DUMMY_G = _N * _NCH
_DUMMY_A = _NH

_ROWS_TC = 2504
_ROWS_TC2 = 2000


def _tc_pool(h, W, b):
  def body(h_ref, w_ref, b_ref, o_ref):
    i = pl.program_id(0)
    y = lax.dot_general(h_ref[...], w_ref[...], (((1,), (1,)), ((), ())),
                        preferred_element_type=jnp.float32,
                        precision=lax.Precision.HIGHEST)
    y = y + b_ref[...]
    row = i * _ROWS_TC + lax.broadcasted_iota(jnp.int32, y.shape, 0)
    o_ref[...] = jnp.where(row < _N, jnp.maximum(y, 0.0), 0.0)

  return pl.pallas_call(
      body,
      out_shape=jax.ShapeDtypeStruct((_NP, _D), jnp.float32),
      grid=(_NP // _ROWS_TC,),
      in_specs=[
          pl.BlockSpec((_ROWS_TC, _D), lambda i: (i, 0)),
          pl.BlockSpec((_D, _D), lambda i: (0, 0)),
          pl.BlockSpec((1, _D), lambda i: (0, 0)),
      ],
      out_specs=pl.BlockSpec((_ROWS_TC, _D), lambda i: (i, 0)),
  )(h, W, b.reshape(1, _D))


def _tc_combine(h, agg, Ws, Wn, bias, g, be):
  def body(h_ref, a_ref, ws_ref, wn_ref, b_ref, g_ref, e_ref, o_ref):
    x = lax.dot_general(h_ref[...], ws_ref[...], (((1,), (1,)), ((), ())),
                        preferred_element_type=jnp.float32,
                        precision=lax.Precision.HIGHEST)
    x = x + lax.dot_general(a_ref[...], wn_ref[...], (((1,), (1,)), ((), ())),
                            preferred_element_type=jnp.float32,
                            precision=lax.Precision.HIGHEST)
    x = x + b_ref[...]
    mu = jnp.mean(x, axis=1, keepdims=True)
    xc = x - mu
    var = jnp.mean(xc * xc, axis=1, keepdims=True)
    y = xc * lax.rsqrt(var + _EPS) * g_ref[...] + e_ref[...]
    o_ref[...] = jnp.maximum(y, 0.0)

  return pl.pallas_call(
      body,
      out_shape=jax.ShapeDtypeStruct((_N, _D), jnp.float32),
      grid=(_N // _ROWS_TC2,),
      in_specs=[
          pl.BlockSpec((_ROWS_TC2, _D), lambda i: (i, 0)),
          pl.BlockSpec((_ROWS_TC2, _D), lambda i: (i, 0)),
          pl.BlockSpec((_D, _D), lambda i: (0, 0)),
          pl.BlockSpec((_D, _D), lambda i: (0, 0)),
          pl.BlockSpec((1, _D), lambda i: (0, 0)),
          pl.BlockSpec((1, _D), lambda i: (0, 0)),
          pl.BlockSpec((1, _D), lambda i: (0, 0)),
      ],
      out_specs=pl.BlockSpec((_ROWS_TC2, _D), lambda i: (i, 0)),
  )(h, agg, Ws, Wn, bias.reshape(1, _D), g.reshape(1, _D), be.reshape(1, _D))


def _sc_segment_max(hp_flat, src, dst):
  mesh = plsc.VectorSubcoreMesh(core_axis_name="c", subcore_axis_name="s")

  @functools.partial(
      pl.kernel,
      out_type=jax.ShapeDtypeStruct((_NCH, _N, _L), jnp.float32),
      mesh=mesh,
      compiler_params=pltpu.CompilerParams(use_tc_tiling_on_sc=False),
      scratch_types=[
          pltpu.VMEM((_NHP, _L), jnp.float32),
          pltpu.VMEM((_B,), jnp.int32),
          pltpu.VMEM((_B,), jnp.int32),
          pltpu.VMEM((_B,), jnp.int32),
          pltpu.VMEM((_B,), jnp.int32),
          pltpu.VMEM((_B, _L), jnp.float32),
          pltpu.VMEM_SHARED((_NCH, _B, _L), jnp.float32),
          pltpu.SemaphoreType.DMA,
      ],
  )
  def k(hp_hbm, src_hbm, dst_hbm, out_hbm,
        agg_l, dst_b, src_b, dloc_b, gidx_b, rows_b, shr, gsem):
    cid = lax.axis_index("c")
    sid = lax.axis_index("s")
    chunk = sid % _NCH
    eseg = sid // _NCH
    half = cid
    lo = half * _NH
    ebase = eseg * _EH

    zrow = jnp.zeros((_L,), jnp.float32)
    def zero_agg(i, carry):
      agg_l[i] = zrow
      return carry
    lax.fori_loop(0, _NHP, zero_agg, 0)

    def block(b, carry):
      base = ebase + b * _B
      pltpu.sync_copy(dst_hbm.at[pl.ds(base, _B)], dst_b)
      pltpu.sync_copy(src_hbm.at[pl.ds(base, _B)], src_b)

      def prep(i, carry2):
        d = dst_b[pl.ds(i * _L, _L)]
        s = src_b[pl.ds(i * _L, _L)]
        m = (d >= lo) & (d < lo + _NH)
        dloc_b[pl.ds(i * _L, _L)] = jnp.where(m, d - lo, _DUMMY_A)
        gidx_b[pl.ds(i * _L, _L)] = jnp.where(m, s * _NCH + chunk, _DUMMY_G)
        return carry2
      lax.fori_loop(0, _B // _L, prep, 0)

      for j in range(_B // _GCH):
        pltpu.make_async_copy(
            hp_hbm.at[gidx_b.at[pl.ds(j * _GCH, _GCH)]],
            rows_b.at[pl.ds(j * _GCH, _GCH)],
            gsem).start()
      for j in range(_B // _GCH):
        pltpu.make_async_copy(
            hp_hbm.at[gidx_b.at[pl.ds(j * _GCH, _GCH)]],
            rows_b.at[pl.ds(j * _GCH, _GCH)],
            gsem).wait()

      def accg(gi, carry2):
        dvec = dloc_b[pl.ds(gi * _L, _L)]
        for j in range(_L):
          dd = dvec[j]
          agg_l[dd] = jnp.maximum(agg_l[dd], rows_b[gi * _L + j])
        return carry2
      lax.fori_loop(0, _B // _L, accg, 0)
      return carry

    lax.fori_loop(0, _NBLK, block, 0)

    for p in range(_NHP // _B):
      @pl.when(eseg == 1)
      def _publish(p=p):
        pltpu.sync_copy(agg_l.at[pl.ds(p * _B, _B)], shr.at[chunk])
      plsc.subcore_barrier()

      @pl.when(eseg == 0)
      def _merge(p=p):
        pltpu.sync_copy(shr.at[chunk], rows_b)
        def mg(i, carry):
          r = p * _B + i
          agg_l[r] = jnp.maximum(agg_l[r], rows_b[i])
          return carry
        lax.fori_loop(0, _B, mg, 0)
      plsc.subcore_barrier()

    @pl.when(eseg == 0)
    def _store():
      pltpu.sync_copy(agg_l.at[pl.ds(0, _NH)],
                      out_hbm.at[chunk, pl.ds(lo, _NH)])

  return k(hp_flat, src, dst)


def _layer(h, src, dst, Wp, bp, Ws, Wn, bias, g, be):
  hp = _tc_pool(h, Wp, bp)
  agg3 = _sc_segment_max(hp.reshape(_NP * _NCH, _L), src, dst)
  agg = agg3.transpose(1, 0, 2).reshape(_N, _D)
  return _tc_combine(h, agg, Ws, Wn, bias, g, be)


def kernel(h, edge_index,
           W_pool0, b_pool0, W_self0, W_neigh0, bias0, ln_g0, ln_b0,
           W_pool1, b_pool1, W_self1, W_neigh1, bias1, ln_g1, ln_b1):
  src = edge_index[0]
  dst = edge_index[1]
  h = _layer(h, src, dst, W_pool0, b_pool0, W_self0, W_neigh0, bias0, ln_g0, ln_b0)
  h = _layer(h, src, dst, W_pool1, b_pool1, W_self1, W_neigh1, bias1, ln_g1, ln_b1)
  return h


# speedup vs baseline: 10.0594x; 10.0594x over previous
import functools

import jax
import jax.numpy as jnp
from jax import lax
from jax.experimental import pallas as pl
from jax.experimental.pallas import tpu as pltpu
from jax.experimental.pallas import tpu_sc as plsc

_N = 10000
_E = 320000
_D = 128
_EPS = 1e-5
_L = 16
_NCH = _D // _L
_NH = _N // 2
_NHP = 5120
_EH = _E // 2
_B = 1280
_NBLK = _EH // _B
_GCH = 128

_NP = 10016
_DUMMY_G = _N * _NCH
_DUMMY_A = _NH

_ROWS_TC = 2504
_ROWS_TC2 = 2000


def _tc_pool(h, W, b):
  def body(h_ref, w_ref, b_ref, o_ref):
    i = pl.program_id(0)
    y = lax.dot_general(h_ref[...], w_ref[...], (((1,), (1,)), ((), ())),
                        preferred_element_type=jnp.float32,
                        precision=lax.Precision.HIGHEST)
    y = y + b_ref[...]
    row = i * _ROWS_TC + lax.broadcasted_iota(jnp.int32, y.shape, 0)
    o_ref[...] = jnp.where(row < _N, jnp.maximum(y, 0.0), 0.0)

  return pl.pallas_call(
      body,
      out_shape=jax.ShapeDtypeStruct((_NP, _D), jnp.float32),
      grid=(_NP // _ROWS_TC,),
      in_specs=[
          pl.BlockSpec((_ROWS_TC, _D), lambda i: (i, 0)),
          pl.BlockSpec((_D, _D), lambda i: (0, 0)),
          pl.BlockSpec((1, _D), lambda i: (0, 0)),
      ],
      out_specs=pl.BlockSpec((_ROWS_TC, _D), lambda i: (i, 0)),
  )(h, W, b.reshape(1, _D))


def _tc_combine(h, agg, Ws, Wn, bias, g, be):
  def body(h_ref, a_ref, ws_ref, wn_ref, b_ref, g_ref, e_ref, o_ref):
    x = lax.dot_general(h_ref[...], ws_ref[...], (((1,), (1,)), ((), ())),
                        preferred_element_type=jnp.float32,
                        precision=lax.Precision.HIGHEST)
    x = x + lax.dot_general(a_ref[...], wn_ref[...], (((1,), (1,)), ((), ())),
                            preferred_element_type=jnp.float32,
                            precision=lax.Precision.HIGHEST)
    x = x + b_ref[...]
    mu = jnp.mean(x, axis=1, keepdims=True)
    xc = x - mu
    var = jnp.mean(xc * xc, axis=1, keepdims=True)
    y = xc * lax.rsqrt(var + _EPS) * g_ref[...] + e_ref[...]
    o_ref[...] = jnp.maximum(y, 0.0)

  return pl.pallas_call(
      body,
      out_shape=jax.ShapeDtypeStruct((_N, _D), jnp.float32),
      grid=(_N // _ROWS_TC2,),
      in_specs=[
          pl.BlockSpec((_ROWS_TC2, _D), lambda i: (i, 0)),
          pl.BlockSpec((_ROWS_TC2, _D), lambda i: (i, 0)),
          pl.BlockSpec((_D, _D), lambda i: (0, 0)),
          pl.BlockSpec((_D, _D), lambda i: (0, 0)),
          pl.BlockSpec((1, _D), lambda i: (0, 0)),
          pl.BlockSpec((1, _D), lambda i: (0, 0)),
          pl.BlockSpec((1, _D), lambda i: (0, 0)),
      ],
      out_specs=pl.BlockSpec((_ROWS_TC2, _D), lambda i: (i, 0)),
  )(h, agg, Ws, Wn, bias.reshape(1, _D), g.reshape(1, _D), be.reshape(1, _D))


def _sc_segment_max(hp_flat, src, dst):
  mesh = plsc.VectorSubcoreMesh(core_axis_name="c", subcore_axis_name="s")

  @functools.partial(
      pl.kernel,
      out_type=jax.ShapeDtypeStruct((_NCH, _N, _L), jnp.float32),
      mesh=mesh,
      compiler_params=pltpu.CompilerParams(use_tc_tiling_on_sc=False,
                                           needs_layout_passes=False),
      scratch_types=[
          pltpu.VMEM((_NHP + 8, _L), jnp.float32),
          pltpu.VMEM((_B,), jnp.int32),
          pltpu.VMEM((_B,), jnp.int32),
          pltpu.VMEM((_B + _L,), jnp.int32),
          pltpu.VMEM((_B + _GCH,), jnp.int32),
          pltpu.VMEM((_B + _L, _L), jnp.float32),
          pltpu.VMEM_SHARED((_NCH, _B, _L), jnp.float32),
          pltpu.SemaphoreType.DMA,
      ],
  )
  def k(hp_hbm, src_hbm, dst_hbm, out_hbm,
        agg_l, dst_b, src_b, dloc_b, gidx_b, rows_b, shr, gsem):
    cid = lax.axis_index("c")
    sid = lax.axis_index("s")
    chunk = sid % _NCH
    eseg = sid // _NCH
    half = cid
    lo = half * _NH
    ebase = eseg * _EH
    lanes = lax.iota(jnp.int32, _L)

    zrow = jnp.zeros((_L,), jnp.float32)
    def zero_agg(i, carry):
      agg_l[i] = zrow
      return carry
    lax.fori_loop(0, _NHP + 8, zero_agg, 0)

    zidx = jnp.zeros((_L,), jnp.int32)
    def zero_gidx(i, carry):
      gidx_b[pl.ds(i * _L, _L)] = zidx
      return carry
    lax.fori_loop(0, (_B + _GCH) // _L, zero_gidx, 0)

    def block(b, carry):
      base = ebase + b * _B
      pltpu.sync_copy(dst_hbm.at[pl.ds(base, _B)], dst_b)
      pltpu.sync_copy(src_hbm.at[pl.ds(base, _B)], src_b)

      def prep(i, nvec):
        d = dst_b[pl.ds(i * _L, _L)]
        s = src_b[pl.ds(i * _L, _L)]
        m = (d >= lo) & (d < lo + _NH)
        cs = plsc.cumsum(jnp.where(m, 1, 0))
        tgt = nvec + cs - 1
        plsc.store_scatter(dloc_b, [tgt], d - lo, mask=m)
        plsc.store_scatter(gidx_b, [tgt], s * _NCH + chunk, mask=m)
        return nvec + plsc.all_reduce_population_count(m)
      nvec = lax.fori_loop(0, _B // _L, prep,
                           jnp.zeros((_L,), jnp.int32))
      plsc.store_scatter(dloc_b, [nvec + lanes],
                         jnp.full((_L,), _DUMMY_A, jnp.int32))
      nsc = nvec[0]

      nch = (nsc + _GCH - 1) // _GCH
      def fire(j, carry2):
        pltpu.make_async_copy(
            hp_hbm.at[gidx_b.at[pl.ds(j * _GCH, _GCH)]],
            rows_b.at[pl.ds(j * _GCH, _GCH)],
            gsem).start()
        return carry2
      lax.fori_loop(0, nch, fire, 0)
      def drain(j, carry2):
        pltpu.make_async_copy(
            hp_hbm.at[gidx_b.at[pl.ds(j * _GCH, _GCH)]],
            rows_b.at[pl.ds(j * _GCH, _GCH)],
            gsem).wait()
        return carry2
      lax.fori_loop(0, nch, drain, 0)

      ngrp = (nsc + _L - 1) // _L
      def accg(gi, carry2):
        dvec = dloc_b[pl.ds(gi * _L, _L)]
        erow = gi * _L + lanes
        for kk in range(_L):
          col = (lanes + kk) & (_L - 1)
          a = plsc.load_gather(agg_l, [dvec, col])
          r = plsc.load_gather(rows_b, [erow, col])
          plsc.store_scatter(agg_l, [dvec, col], jnp.maximum(a, r))
        return carry2
      lax.fori_loop(0, ngrp, accg, 0)
      return carry

    lax.fori_loop(0, _NBLK, block, 0)

    for p in range(_NHP // _B):
      @pl.when(eseg == 1)
      def _publish(p=p):
        pltpu.sync_copy(agg_l.at[pl.ds(p * _B, _B)], shr.at[chunk])
      plsc.subcore_barrier()

      @pl.when(eseg == 0)
      def _merge(p=p):
        pltpu.sync_copy(shr.at[chunk], rows_b.at[pl.ds(0, _B)])
        def mg(i, carry):
          r = p * _B + i
          agg_l[r] = jnp.maximum(agg_l[r], rows_b[i])
          return carry
        lax.fori_loop(0, _B, mg, 0)
      plsc.subcore_barrier()

    @pl.when(eseg == 0)
    def _store():
      pltpu.sync_copy(agg_l.at[pl.ds(0, _NH)],
                      out_hbm.at[chunk, pl.ds(lo, _NH)])

  return k(hp_flat, src, dst)


def _layer(h, src, dst, Wp, bp, Ws, Wn, bias, g, be):
  hp = _tc_pool(h, Wp, bp)
  agg3 = _sc_segment_max(hp.reshape(_NP * _NCH, _L), src, dst)
  agg = agg3.transpose(1, 0, 2).reshape(_N, _D)
  return _tc_combine(h, agg, Ws, Wn, bias, g, be)


def kernel(h, edge_index,
           W_pool0, b_pool0, W_self0, W_neigh0, bias0, ln_g0, ln_b0,
           W_pool1, b_pool1, W_self1, W_neigh1, bias1, ln_g1, ln_b1):
  src = edge_index[0]
  dst = edge_index[1]
  h = _layer(h, src, dst, W_pool0, b_pool0, W_self0, W_neigh0, bias0, ln_g0, ln_b0)
  h = _layer(h, src, dst, W_pool1, b_pool1, W_self1, W_neigh1, bias1, ln_g1, ln_b1)
  return h
